# Initial kernel scaffold; baseline (speedup 1.0000x reference)
import functools

import jax
import jax.numpy as jnp
from jax import lax
from jax.experimental import pallas as pl
from jax.experimental.pallas import tpu as pltpu
from jax.experimental.pallas import tpu_sc as plsc

N = 10000
D = 128
E = 320000
NC = 2
NS = 16
EPW = E // NS
CH = 80
NCHUNK = EPW // CH
RPW = N // NS
ZR = 125
B = 4096

_MESH = plsc.VectorSubcoreMesh(
    core_axis_name="c", subcore_axis_name="s", num_cores=NC, num_subcores=NS)


@functools.partial(
    pl.kernel,
    out_type=jax.ShapeDtypeStruct((NC, N, D), jnp.float32),
    mesh=_MESH,
    scratch_types=[
        pltpu.VMEM((NCHUNK, CH), jnp.int32),
        pltpu.VMEM((NCHUNK, CH), jnp.int32),
        pltpu.VMEM((NCHUNK, CH), jnp.float32),
        pltpu.VMEM((CH, D), jnp.float32),
        pltpu.VMEM((ZR, D), jnp.float32),
        pltpu.VMEM_SHARED((N, D), jnp.float32),
        pltpu.SemaphoreType.DMA,
    ],
)
def _sc_aggregate(x_hbm, src_hbm, dst_hbm, w_hbm, out_hbm,
                  si_v, di_v, wv_v, rows_v, stage_v, acc_sh, sem):
  c = lax.axis_index("c")
  s = lax.axis_index("s")

  def zrow(i, carry):
    for j in range(D // 16):
      stage_v[i, pl.ds(j * 16, 16)] = jnp.zeros((16,), jnp.float32)
    return carry
  lax.fori_loop(0, ZR, zrow, 0)
  r0 = s * RPW
  for z in range(RPW // ZR):
    pltpu.sync_copy(stage_v, acc_sh.at[pl.ds(r0 + z * ZR, ZR)])
  plsc.subcore_barrier()

  pltpu.sync_copy(src_hbm.at[s], si_v)
  pltpu.sync_copy(dst_hbm.at[s], di_v)
  pltpu.sync_copy(w_hbm.at[c, s], wv_v)

  def chunk(k, carry):
    pltpu.async_copy(x_hbm.at[si_v.at[k]], rows_v, sem).wait()

    def col(j, carry2):
      cids = jnp.full((16,), j, dtype=jnp.int32)
      for g in range(CH // 16):
        eids = jnp.int32(g * 16) + lax.iota(jnp.int32, 16)
        vals = plsc.load_gather(rows_v, [eids, cids])
        wg = wv_v[k, pl.ds(g * 16, 16)]
        plsc.store_scatter(rows_v, [eids, cids], vals * wg)
      return carry2
    lax.fori_loop(0, D, col, 0)

    pltpu.sync_copy(rows_v, acc_sh.at[di_v.at[k]], add=True)
    return carry
  lax.fori_loop(0, NCHUNK, chunk, 0)

  plsc.subcore_barrier()
  for z in range(RPW // ZR):
    rr = r0 + z * ZR
    pltpu.sync_copy(acc_sh.at[pl.ds(rr, ZR)], out_hbm.at[c, pl.ds(rr, ZR)])


_GB = B * 2 // (NC * NS)


@functools.partial(
    pl.kernel,
    out_type=jax.ShapeDtypeStruct((2 * B, D), jnp.float32),
    mesh=_MESH,
    scratch_types=[
        pltpu.VMEM((128,), jnp.int32),
        pltpu.VMEM((128, D), jnp.float32),
        pltpu.SemaphoreType.DMA,
    ],
)
def _sc_gather_rows(x_hbm, idx_hbm, out_hbm, idx_v, rows_v, sem):
  c = lax.axis_index("c")
  s = lax.axis_index("s")
  base = (s * NC + c) * _GB
  for t in range(_GB // 128):
    off = base + t * 128
    pltpu.sync_copy(idx_hbm.at[pl.ds(off, 128)], idx_v)
    pltpu.async_copy(x_hbm.at[idx_v], rows_v, sem).wait()
    pltpu.sync_copy(rows_v, out_hbm.at[pl.ds(off, 128)])


def _wsel_call(w2d, t2d):
  def body(w_ref, t_ref, o_ref):
    w = w_ref[...]
    t = t_ref[...]
    o_ref[0] = jnp.where(t == 0, w, 0.0)
    o_ref[1] = jnp.where(t == 1, w, 0.0)
  return pl.pallas_call(
      body,
      out_shape=jax.ShapeDtypeStruct((2, E // 128, 128), jnp.float32),
  )(w2d, t2d)


RB = 1000


def _tc_layer_call(A, x, relw, rootw, bias, g, b2, do_ln):
  def body(a_ref, x_ref, rw_ref, rootw_ref, bias_ref, g_ref, b2_ref, o_ref):
    y = jnp.dot(a_ref[0], rw_ref[0], preferred_element_type=jnp.float32)
    y = y + jnp.dot(a_ref[1], rw_ref[1], preferred_element_type=jnp.float32)
    y = y + jnp.dot(x_ref[...], rootw_ref[...],
                    preferred_element_type=jnp.float32)
    y = y + bias_ref[...]
    if do_ln:
      y = jnp.maximum(y, 0.0)
      m = jnp.mean(y, axis=-1, keepdims=True)
      yc = y - m
      v = jnp.mean(yc * yc, axis=-1, keepdims=True)
      y = yc * lax.rsqrt(v + 1e-5) * g_ref[...] + b2_ref[...]
    o_ref[...] = y
  return pl.pallas_call(
      body,
      grid=(N // RB,),
      in_specs=[
          pl.BlockSpec((2, RB, D), lambda i: (0, i, 0)),
          pl.BlockSpec((RB, D), lambda i: (i, 0)),
          pl.BlockSpec((2, D, D), lambda i: (0, 0, 0)),
          pl.BlockSpec((D, D), lambda i: (0, 0)),
          pl.BlockSpec((1, D), lambda i: (0, 0)),
          pl.BlockSpec((1, D), lambda i: (0, 0)),
          pl.BlockSpec((1, D), lambda i: (0, 0)),
      ],
      out_specs=pl.BlockSpec((RB, D), lambda i: (i, 0)),
      out_shape=jax.ShapeDtypeStruct((N, D), jnp.float32),
  )(A, x, relw, rootw, bias, g, b2)


def _tc_head_call(ui, w0, b0, w1, b1, w2, b2, owt, ob):
  def body(ui_ref, w0_ref, b0_ref, w1_ref, b1_ref, w2_ref, b2_ref,
           ow_ref, ob_ref, o_ref):
    u = ui_ref[:B]
    it = ui_ref[B:]
    h = (jnp.dot(u, w0_ref[:D], preferred_element_type=jnp.float32)
         + jnp.dot(it, w0_ref[D:], preferred_element_type=jnp.float32)
         + b0_ref[...])
    h = jnp.maximum(h, 0.0)
    h = jnp.maximum(
        jnp.dot(h, w1_ref[...], preferred_element_type=jnp.float32)
        + b1_ref[...], 0.0)
    h = jnp.maximum(
        jnp.dot(h, w2_ref[...], preferred_element_type=jnp.float32)
        + b2_ref[...], 0.0)
    nu = jnp.maximum(jnp.sqrt(jnp.sum(u * u, axis=-1, keepdims=True)), 1e-12)
    ni = jnp.maximum(jnp.sqrt(jnp.sum(it * it, axis=-1, keepdims=True)),
                     1e-12)
    gmf = (u / nu) * (it / ni)
    logit = (jnp.sum(gmf * ow_ref[:, :D], axis=-1, keepdims=True)
             + jnp.sum(h * ow_ref[:, D:], axis=-1, keepdims=True)
             + ob_ref[...])
    o_ref[...] = jax.nn.sigmoid(logit)
  return pl.pallas_call(
      body,
      out_shape=jax.ShapeDtypeStruct((B, 1), jnp.float32),
  )(ui, w0, b0, w1, b1, w2, b2, owt, ob)


def kernel(user_indices, item_indices, edge_index, edge_type, edge_weight,
           emb_table, rel_w0, rel_w1, rel_w2, root_w0, root_w1, root_w2,
           bias0, bias1, bias2, ln1_g, ln1_b, ln2_g, ln2_b,
           mlp_w0, mlp_b0, mlp_w1, mlp_b1, mlp_w2, mlp_b2, out_w, out_b):
  src3 = edge_index[0].astype(jnp.int32).reshape(NS, NCHUNK, CH)
  dst3 = edge_index[1].astype(jnp.int32).reshape(NS, NCHUNK, CH)
  t2d = edge_type.astype(jnp.int32).reshape(E // 128, 128)
  w2d = edge_weight.reshape(E // 128, 128)
  wsel = _wsel_call(w2d, t2d).reshape(NC, NS, NCHUNK, CH)

  x = emb_table
  layers = [
      (rel_w0, root_w0, bias0, ln1_g, ln1_b, True),
      (rel_w1, root_w1, bias1, ln2_g, ln2_b, True),
      (rel_w2, root_w2, bias2, ln2_g, ln2_b, False),
  ]
  for relw, rootw, bias, g, b2, do_ln in layers:
    A = _sc_aggregate(x, src3, dst3, wsel)
    x = _tc_layer_call(A, x, relw, rootw.reshape(D, D),
                       bias.reshape(1, D), g.reshape(1, D),
                       b2.reshape(1, D), do_ln)

  idx = jnp.concatenate([user_indices, item_indices]).astype(jnp.int32)
  ui = _sc_gather_rows(x, idx)
  out = _tc_head_call(
      ui, mlp_w0, mlp_b0.reshape(1, -1), mlp_w1, mlp_b1.reshape(1, -1),
      mlp_w2, mlp_b2.reshape(1, -1), out_w.reshape(1, -1),
      out_b.reshape(1, 1))
  return out.reshape(B)


# speedup vs baseline: 2.5011x; 2.5011x over previous
import functools

import jax
import jax.numpy as jnp
from jax import lax
from jax.experimental import pallas as pl
from jax.experimental.pallas import tpu as pltpu
from jax.experimental.pallas import tpu_sc as plsc

N = 10000
D = 128
E = 320000
NC = 2
NS = 16
EPW = E // NS
CH = 80
NCHUNK = EPW // CH
NP = 10240
RPW = NP // NS
ZR = 128
B = 4096

_MESH = plsc.VectorSubcoreMesh(
    core_axis_name="c", subcore_axis_name="s", num_cores=NC, num_subcores=NS)


@functools.partial(
    pl.kernel,
    out_type=jax.ShapeDtypeStruct((NC, NP, D), jnp.float32),
    mesh=_MESH,
    scratch_types=[
        pltpu.VMEM((CH,), jnp.int32),
        pltpu.VMEM((CH,), jnp.int32),
        pltpu.VMEM((CH,), jnp.float32),
        pltpu.VMEM((CH, D), jnp.float32),
        pltpu.VMEM((ZR, D), jnp.float32),
        pltpu.VMEM_SHARED((NP, D), jnp.float32),
        pltpu.SemaphoreType.DMA,
    ],
)
def _sc_aggregate(x_hbm, src_hbm, dst_hbm, w_hbm, out_hbm,
                  si_v, di_v, wv_v, rows_v, stage_v, acc_sh, sem):
  c = lax.axis_index("c")
  s = lax.axis_index("s")

  def zrow(i, carry):
    for j in range(D // 16):
      stage_v[i, pl.ds(j * 16, 16)] = jnp.zeros((16,), jnp.float32)
    return carry
  lax.fori_loop(0, ZR, zrow, 0)
  r0 = s * RPW
  for z in range(RPW // ZR):
    pltpu.sync_copy(stage_v, acc_sh.at[pl.ds(r0 + z * ZR, ZR)])
  plsc.subcore_barrier()

  e0 = s * EPW

  def chunk(k, carry):
    off = e0 + k * CH
    pltpu.sync_copy(src_hbm.at[pl.ds(off, CH)], si_v)
    pltpu.sync_copy(dst_hbm.at[pl.ds(off, CH)], di_v)
    pltpu.sync_copy(w_hbm.at[pl.ds(c * E + off, CH)], wv_v)
    pltpu.async_copy(x_hbm.at[si_v], rows_v, sem).wait()

    def grp(g, carry2):
      wg = wv_v[pl.ds(g * 16, 16)]
      for i in range(16):
        ws = wg[i]
        e = g * 16 + i
        for j in range(D // 16):
          sl = pl.ds(j * 16, 16)
          rows_v[e, sl] = rows_v[e, sl] * ws
      return carry2
    lax.fori_loop(0, CH // 16, grp, 0)

    pltpu.sync_copy(rows_v, acc_sh.at[di_v], add=True)
    return carry
  lax.fori_loop(0, NCHUNK, chunk, 0)

  plsc.subcore_barrier()
  for z in range(RPW // ZR):
    rr = r0 + z * ZR
    pltpu.sync_copy(acc_sh.at[pl.ds(rr, ZR)], out_hbm.at[c, pl.ds(rr, ZR)])


_GB = B * 2 // (NC * NS)


@functools.partial(
    pl.kernel,
    out_type=jax.ShapeDtypeStruct((2 * B, D), jnp.float32),
    mesh=_MESH,
    scratch_types=[
        pltpu.VMEM((128,), jnp.int32),
        pltpu.VMEM((128, D), jnp.float32),
        pltpu.SemaphoreType.DMA,
    ],
)
def _sc_gather_rows(x_hbm, idx_hbm, out_hbm, idx_v, rows_v, sem):
  c = lax.axis_index("c")
  s = lax.axis_index("s")
  base = (s * NC + c) * _GB
  for t in range(_GB // 128):
    off = base + t * 128
    pltpu.sync_copy(idx_hbm.at[pl.ds(off, 128)], idx_v)
    pltpu.async_copy(x_hbm.at[idx_v], rows_v, sem).wait()
    pltpu.sync_copy(rows_v, out_hbm.at[pl.ds(off, 128)])


def _wsel_call(w2d, t2d):
  def body(w_ref, t_ref, o_ref):
    w = w_ref[...]
    t = t_ref[...]
    o_ref[0] = jnp.where(t == 0, w, 0.0)
    o_ref[1] = jnp.where(t == 1, w, 0.0)
  return pl.pallas_call(
      body,
      out_shape=jax.ShapeDtypeStruct((2, E // 128, 128), jnp.float32),
  )(w2d, t2d)


RB = 1000


def _tc_layer_call(A, x, relw, rootw, bias, g, b2, do_ln):
  def body(a_ref, x_ref, rw_ref, rootw_ref, bias_ref, g_ref, b2_ref, o_ref):
    y = jnp.dot(a_ref[0], rw_ref[0], preferred_element_type=jnp.float32)
    y = y + jnp.dot(a_ref[1], rw_ref[1], preferred_element_type=jnp.float32)
    y = y + jnp.dot(x_ref[...], rootw_ref[...],
                    preferred_element_type=jnp.float32)
    y = y + bias_ref[...]
    if do_ln:
      y = jnp.maximum(y, 0.0)
      m = jnp.mean(y, axis=-1, keepdims=True)
      yc = y - m
      v = jnp.mean(yc * yc, axis=-1, keepdims=True)
      y = yc * lax.rsqrt(v + 1e-5) * g_ref[...] + b2_ref[...]
    o_ref[...] = y
  return pl.pallas_call(
      body,
      grid=(N // RB,),
      in_specs=[
          pl.BlockSpec((2, RB, D), lambda i: (0, i, 0)),
          pl.BlockSpec((RB, D), lambda i: (i, 0)),
          pl.BlockSpec((2, D, D), lambda i: (0, 0, 0)),
          pl.BlockSpec((D, D), lambda i: (0, 0)),
          pl.BlockSpec((1, D), lambda i: (0, 0)),
          pl.BlockSpec((1, D), lambda i: (0, 0)),
          pl.BlockSpec((1, D), lambda i: (0, 0)),
      ],
      out_specs=pl.BlockSpec((RB, D), lambda i: (i, 0)),
      out_shape=jax.ShapeDtypeStruct((N, D), jnp.float32),
  )(A, x, relw, rootw, bias, g, b2)


def _tc_head_call(ui, w0, b0, w1, b1, w2, b2, owt, ob):
  def body(ui_ref, w0_ref, b0_ref, w1_ref, b1_ref, w2_ref, b2_ref,
           ow_ref, ob_ref, o_ref):
    u = ui_ref[:B]
    it = ui_ref[B:]
    h = (jnp.dot(u, w0_ref[:D], preferred_element_type=jnp.float32)
         + jnp.dot(it, w0_ref[D:], preferred_element_type=jnp.float32)
         + b0_ref[...])
    h = jnp.maximum(h, 0.0)
    h = jnp.maximum(
        jnp.dot(h, w1_ref[...], preferred_element_type=jnp.float32)
        + b1_ref[...], 0.0)
    h = jnp.maximum(
        jnp.dot(h, w2_ref[...], preferred_element_type=jnp.float32)
        + b2_ref[...], 0.0)
    nu = jnp.maximum(jnp.sqrt(jnp.sum(u * u, axis=-1, keepdims=True)), 1e-12)
    ni = jnp.maximum(jnp.sqrt(jnp.sum(it * it, axis=-1, keepdims=True)),
                     1e-12)
    gmf = (u / nu) * (it / ni)
    logit = (jnp.sum(gmf * ow_ref[:, :D], axis=-1, keepdims=True)
             + jnp.sum(h * ow_ref[:, D:], axis=-1, keepdims=True)
             + ob_ref[...])
    o_ref[...] = jax.nn.sigmoid(logit)
  return pl.pallas_call(
      body,
      out_shape=jax.ShapeDtypeStruct((B, 1), jnp.float32),
  )(ui, w0, b0, w1, b1, w2, b2, owt, ob)


def kernel(user_indices, item_indices, edge_index, edge_type, edge_weight,
           emb_table, rel_w0, rel_w1, rel_w2, root_w0, root_w1, root_w2,
           bias0, bias1, bias2, ln1_g, ln1_b, ln2_g, ln2_b,
           mlp_w0, mlp_b0, mlp_w1, mlp_b1, mlp_w2, mlp_b2, out_w, out_b):
  src1 = edge_index[0].astype(jnp.int32)
  dst1 = edge_index[1].astype(jnp.int32)
  t2d = edge_type.astype(jnp.int32).reshape(E // 128, 128)
  w2d = edge_weight.reshape(E // 128, 128)
  wsel = _wsel_call(w2d, t2d).reshape(NC * E)

  x = emb_table
  layers = [
      (rel_w0, root_w0, bias0, ln1_g, ln1_b, True),
      (rel_w1, root_w1, bias1, ln2_g, ln2_b, True),
      (rel_w2, root_w2, bias2, ln2_g, ln2_b, False),
  ]
  for relw, rootw, bias, g, b2, do_ln in layers:
    A = _sc_aggregate(x, src1, dst1, wsel)
    x = _tc_layer_call(A, x, relw, rootw.reshape(D, D),
                       bias.reshape(1, D), g.reshape(1, D),
                       b2.reshape(1, D), do_ln)

  idx = jnp.concatenate([user_indices, item_indices]).astype(jnp.int32)
  ui = _sc_gather_rows(x, idx)
  out = _tc_head_call(
      ui, mlp_w0, mlp_b0.reshape(1, -1), mlp_w1, mlp_b1.reshape(1, -1),
      mlp_w2, mlp_b2.reshape(1, -1), out_w.reshape(1, -1),
      out_b.reshape(1, 1))
  return out.reshape(B)


# speedup vs baseline: 3.9909x; 1.5957x over previous
import functools

import jax
import jax.numpy as jnp
from jax import lax
from jax.experimental import pallas as pl
from jax.experimental.pallas import tpu as pltpu
from jax.experimental.pallas import tpu_sc as plsc

N = 10000
D = 128
E = 320000
NC = 2
NS = 16
EPW = E // NS
CH = 80
NCHUNK = EPW // CH
NP = 10240
RPW = NP // NS
ZR = 128
B = 4096

_MESH = plsc.VectorSubcoreMesh(
    core_axis_name="c", subcore_axis_name="s", num_cores=NC, num_subcores=NS)


@functools.partial(
    pl.kernel,
    out_type=jax.ShapeDtypeStruct((NC, NP, D), jnp.float32),
    mesh=_MESH,
    scratch_types=[
        pltpu.VMEM((CH,), jnp.int32),
        pltpu.VMEM((CH,), jnp.int32),
        pltpu.VMEM((CH,), jnp.float32),
        pltpu.VMEM((CH, D), jnp.float32),
        pltpu.VMEM((CH,), jnp.int32),
        pltpu.VMEM((CH,), jnp.int32),
        pltpu.VMEM((CH,), jnp.float32),
        pltpu.VMEM((CH, D), jnp.float32),
        pltpu.VMEM((ZR, D), jnp.float32),
        pltpu.VMEM_SHARED((NP, D), jnp.float32),
        pltpu.SemaphoreType.DMA,
        pltpu.SemaphoreType.DMA,
        pltpu.SemaphoreType.DMA,
        pltpu.SemaphoreType.DMA,
    ],
)
def _sc_aggregate(x_hbm, src_hbm, dst_hbm, w_hbm, out_hbm,
                  si0, di0, wv0, rows0, si1, di1, wv1, rows1,
                  stage_v, acc_sh, gsem0, gsem1, msem0, msem1):
  c = lax.axis_index("c")
  s = lax.axis_index("s")

  def zrow(i, carry):
    for j in range(D // 16):
      stage_v[i, pl.ds(j * 16, 16)] = jnp.zeros((16,), jnp.float32)
    return carry
  lax.fori_loop(0, ZR, zrow, 0)
  r0 = s * RPW
  for z in range(RPW // ZR):
    pltpu.sync_copy(stage_v, acc_sh.at[pl.ds(r0 + z * ZR, ZR)])
  plsc.subcore_barrier()

  e0 = s * EPW
  sets = ((si0, di0, wv0, rows0, gsem0, msem0),
          (si1, di1, wv1, rows1, gsem1, msem1))

  def issue_meta(k, st):
    si_r, di_r, wv_r, _, _, msem = st
    off = e0 + k * CH
    pltpu.async_copy(src_hbm.at[pl.ds(off, CH)], si_r, msem)
    pltpu.async_copy(dst_hbm.at[pl.ds(off, CH)], di_r, msem)
    pltpu.async_copy(w_hbm.at[pl.ds(c * E + off, CH)], wv_r, msem)

  def wait_meta(st):
    si_r, di_r, wv_r, _, _, msem = st
    pltpu.make_async_copy(src_hbm.at[pl.ds(0, CH)], si_r, msem).wait()
    pltpu.make_async_copy(dst_hbm.at[pl.ds(0, CH)], di_r, msem).wait()
    pltpu.make_async_copy(w_hbm.at[pl.ds(0, CH)], wv_r, msem).wait()

  def issue_gather(st):
    si_r, _, _, rows_r, gsem, _ = st
    pltpu.async_copy(x_hbm.at[si_r], rows_r, gsem)

  def wait_gather(st):
    si_r, _, _, rows_r, gsem, _ = st
    pltpu.make_async_copy(x_hbm.at[si_r], rows_r, gsem).wait()

  def scale(st):
    _, _, wv_r, rows_r, _, _ = st
    def grp(g, carry2):
      wg = wv_r[pl.ds(g * 16, 16)]
      for i in range(16):
        ws = wg[i]
        e = g * 16 + i
        for j in range(D // 16):
          sl = pl.ds(j * 16, 16)
          rows_r[e, sl] = rows_r[e, sl] * ws
      return carry2
    lax.fori_loop(0, CH // 16, grp, 0)

  issue_meta(0, sets[0])
  wait_meta(sets[0])
  issue_gather(sets[0])
  issue_meta(1, sets[1])

  def pair(p, carry):
    for h in range(2):
      k = 2 * p + h
      sA = sets[h]
      sB = sets[1 - h]
      wait_gather(sA)
      scale(sA)
      pltpu.sync_copy(sA[3], acc_sh.at[sA[1]], add=True)
      @pl.when(k + 2 < NCHUNK)
      def _():
        issue_meta(k + 2, sA)
      @pl.when(k + 1 < NCHUNK)
      def _():
        wait_meta(sB)
        issue_gather(sB)
    return carry
  lax.fori_loop(0, NCHUNK // 2, pair, 0)

  plsc.subcore_barrier()
  for z in range(RPW // ZR):
    rr = r0 + z * ZR
    pltpu.sync_copy(acc_sh.at[pl.ds(rr, ZR)], out_hbm.at[c, pl.ds(rr, ZR)])


_GB = B * 2 // (NC * NS)


@functools.partial(
    pl.kernel,
    out_type=jax.ShapeDtypeStruct((2 * B, D), jnp.float32),
    mesh=_MESH,
    scratch_types=[
        pltpu.VMEM((128,), jnp.int32),
        pltpu.VMEM((128, D), jnp.float32),
        pltpu.SemaphoreType.DMA,
    ],
)
def _sc_gather_rows(x_hbm, idx_hbm, out_hbm, idx_v, rows_v, sem):
  c = lax.axis_index("c")
  s = lax.axis_index("s")
  base = (s * NC + c) * _GB
  for t in range(_GB // 128):
    off = base + t * 128
    pltpu.sync_copy(idx_hbm.at[pl.ds(off, 128)], idx_v)
    pltpu.async_copy(x_hbm.at[idx_v], rows_v, sem).wait()
    pltpu.sync_copy(rows_v, out_hbm.at[pl.ds(off, 128)])


def _wsel_call(w2d, t2d):
  def body(w_ref, t_ref, o_ref):
    w = w_ref[...]
    t = t_ref[...]
    o_ref[0] = jnp.where(t == 0, w, 0.0)
    o_ref[1] = jnp.where(t == 1, w, 0.0)
  return pl.pallas_call(
      body,
      out_shape=jax.ShapeDtypeStruct((2, E // 128, 128), jnp.float32),
  )(w2d, t2d)


RB = 1000


def _tc_layer_call(A, x, relw, rootw, bias, g, b2, do_ln):
  def body(a_ref, x_ref, rw_ref, rootw_ref, bias_ref, g_ref, b2_ref, o_ref):
    y = jnp.dot(a_ref[0], rw_ref[0], preferred_element_type=jnp.float32)
    y = y + jnp.dot(a_ref[1], rw_ref[1], preferred_element_type=jnp.float32)
    y = y + jnp.dot(x_ref[...], rootw_ref[...],
                    preferred_element_type=jnp.float32)
    y = y + bias_ref[...]
    if do_ln:
      y = jnp.maximum(y, 0.0)
      m = jnp.mean(y, axis=-1, keepdims=True)
      yc = y - m
      v = jnp.mean(yc * yc, axis=-1, keepdims=True)
      y = yc * lax.rsqrt(v + 1e-5) * g_ref[...] + b2_ref[...]
    o_ref[...] = y
  return pl.pallas_call(
      body,
      grid=(N // RB,),
      in_specs=[
          pl.BlockSpec((2, RB, D), lambda i: (0, i, 0)),
          pl.BlockSpec((RB, D), lambda i: (i, 0)),
          pl.BlockSpec((2, D, D), lambda i: (0, 0, 0)),
          pl.BlockSpec((D, D), lambda i: (0, 0)),
          pl.BlockSpec((1, D), lambda i: (0, 0)),
          pl.BlockSpec((1, D), lambda i: (0, 0)),
          pl.BlockSpec((1, D), lambda i: (0, 0)),
      ],
      out_specs=pl.BlockSpec((RB, D), lambda i: (i, 0)),
      out_shape=jax.ShapeDtypeStruct((N, D), jnp.float32),
  )(A, x, relw, rootw, bias, g, b2)


def _tc_head_call(ui, w0, b0, w1, b1, w2, b2, owt, ob):
  def body(ui_ref, w0_ref, b0_ref, w1_ref, b1_ref, w2_ref, b2_ref,
           ow_ref, ob_ref, o_ref):
    u = ui_ref[:B]
    it = ui_ref[B:]
    h = (jnp.dot(u, w0_ref[:D], preferred_element_type=jnp.float32)
         + jnp.dot(it, w0_ref[D:], preferred_element_type=jnp.float32)
         + b0_ref[...])
    h = jnp.maximum(h, 0.0)
    h = jnp.maximum(
        jnp.dot(h, w1_ref[...], preferred_element_type=jnp.float32)
        + b1_ref[...], 0.0)
    h = jnp.maximum(
        jnp.dot(h, w2_ref[...], preferred_element_type=jnp.float32)
        + b2_ref[...], 0.0)
    nu = jnp.maximum(jnp.sqrt(jnp.sum(u * u, axis=-1, keepdims=True)), 1e-12)
    ni = jnp.maximum(jnp.sqrt(jnp.sum(it * it, axis=-1, keepdims=True)),
                     1e-12)
    gmf = (u / nu) * (it / ni)
    logit = (jnp.sum(gmf * ow_ref[:, :D], axis=-1, keepdims=True)
             + jnp.sum(h * ow_ref[:, D:], axis=-1, keepdims=True)
             + ob_ref[...])
    o_ref[...] = jax.nn.sigmoid(logit)
  return pl.pallas_call(
      body,
      out_shape=jax.ShapeDtypeStruct((B, 1), jnp.float32),
  )(ui, w0, b0, w1, b1, w2, b2, owt, ob)


def kernel(user_indices, item_indices, edge_index, edge_type, edge_weight,
           emb_table, rel_w0, rel_w1, rel_w2, root_w0, root_w1, root_w2,
           bias0, bias1, bias2, ln1_g, ln1_b, ln2_g, ln2_b,
           mlp_w0, mlp_b0, mlp_w1, mlp_b1, mlp_w2, mlp_b2, out_w, out_b):
  src1 = edge_index[0].astype(jnp.int32)
  dst1 = edge_index[1].astype(jnp.int32)
  t2d = edge_type.astype(jnp.int32).reshape(E // 128, 128)
  w2d = edge_weight.reshape(E // 128, 128)
  wsel = _wsel_call(w2d, t2d).reshape(NC * E)

  x = emb_table
  layers = [
      (rel_w0, root_w0, bias0, ln1_g, ln1_b, True),
      (rel_w1, root_w1, bias1, ln2_g, ln2_b, True),
      (rel_w2, root_w2, bias2, ln2_g, ln2_b, False),
  ]
  for relw, rootw, bias, g, b2, do_ln in layers:
    A = _sc_aggregate(x, src1, dst1, wsel)
    x = _tc_layer_call(A, x, relw, rootw.reshape(D, D),
                       bias.reshape(1, D), g.reshape(1, D),
                       b2.reshape(1, D), do_ln)

  idx = jnp.concatenate([user_indices, item_indices]).astype(jnp.int32)
  ui = _sc_gather_rows(x, idx)
  out = _tc_head_call(
      ui, mlp_w0, mlp_b0.reshape(1, -1), mlp_w1, mlp_b1.reshape(1, -1),
      mlp_w2, mlp_b2.reshape(1, -1), out_w.reshape(1, -1),
      out_b.reshape(1, 1))
  return out.reshape(B)


# speedup vs baseline: 3.9910x; 1.0000x over previous
import functools

import jax
import jax.numpy as jnp
from jax import lax
from jax.experimental import pallas as pl
from jax.experimental.pallas import tpu as pltpu
from jax.experimental.pallas import tpu_sc as plsc

N = 10000
D = 128
E = 320000
NC = 2
NS = 16
EPW = E // NS
CH = 80
NCHUNK = EPW // CH
NP = 10240
RPW = NP // NS
ZR = 128
B = 4096

_MESH = plsc.VectorSubcoreMesh(
    core_axis_name="c", subcore_axis_name="s", num_cores=NC, num_subcores=NS)


@functools.partial(
    pl.kernel,
    out_type=jax.ShapeDtypeStruct((NC, NP, D), jnp.float32),
    mesh=_MESH,
    scratch_types=[
        pltpu.VMEM((CH,), jnp.int32),
        pltpu.VMEM((CH,), jnp.int32),
        pltpu.VMEM((CH,), jnp.float32),
        pltpu.VMEM((CH, D), jnp.float32),
        pltpu.VMEM((CH,), jnp.int32),
        pltpu.VMEM((CH,), jnp.int32),
        pltpu.VMEM((CH,), jnp.float32),
        pltpu.VMEM((CH, D), jnp.float32),
        pltpu.VMEM((ZR, D), jnp.float32),
        pltpu.VMEM_SHARED((NP, D), jnp.float32),
        pltpu.SemaphoreType.DMA,
        pltpu.SemaphoreType.DMA,
        pltpu.SemaphoreType.DMA,
        pltpu.SemaphoreType.DMA,
        pltpu.SemaphoreType.DMA,
        pltpu.SemaphoreType.DMA,
    ],
)
def _sc_aggregate(x_hbm, src_hbm, dst_hbm, w_hbm, out_hbm,
                  si0, di0, wv0, rows0, si1, di1, wv1, rows1,
                  stage_v, acc_sh, gsem0, gsem1, msem0, msem1, ssem0, ssem1):
  c = lax.axis_index("c")
  s = lax.axis_index("s")

  def zrow(i, carry):
    for j in range(D // 16):
      stage_v[i, pl.ds(j * 16, 16)] = jnp.zeros((16,), jnp.float32)
    return carry
  lax.fori_loop(0, ZR, zrow, 0)
  r0 = s * RPW
  for z in range(RPW // ZR):
    pltpu.sync_copy(stage_v, acc_sh.at[pl.ds(r0 + z * ZR, ZR)])
  plsc.subcore_barrier()

  e0 = s * EPW
  sets = ((si0, di0, wv0, rows0, gsem0, msem0, ssem0),
          (si1, di1, wv1, rows1, gsem1, msem1, ssem1))

  def issue_meta(k, st):
    si_r, di_r, wv_r, _, _, msem, _ = st
    off = e0 + k * CH
    pltpu.async_copy(src_hbm.at[pl.ds(off, CH)], si_r, msem)
    pltpu.async_copy(dst_hbm.at[pl.ds(off, CH)], di_r, msem)
    pltpu.async_copy(w_hbm.at[pl.ds(c * E + off, CH)], wv_r, msem)

  def wait_meta(st):
    si_r, di_r, wv_r, _, _, msem, _ = st
    pltpu.make_async_copy(src_hbm.at[pl.ds(0, CH)], si_r, msem).wait()
    pltpu.make_async_copy(dst_hbm.at[pl.ds(0, CH)], di_r, msem).wait()
    pltpu.make_async_copy(w_hbm.at[pl.ds(0, CH)], wv_r, msem).wait()

  def issue_gather(st):
    si_r, _, _, rows_r, gsem, _, _ = st
    pltpu.async_copy(x_hbm.at[si_r], rows_r, gsem)

  def wait_gather(st):
    si_r, _, _, rows_r, gsem, _, _ = st
    pltpu.make_async_copy(x_hbm.at[si_r], rows_r, gsem).wait()

  def issue_scatter(st):
    _, di_r, _, rows_r, _, _, ssem = st
    pltpu.async_copy(rows_r, acc_sh.at[di_r], ssem, add=True)

  def wait_scatter(st):
    _, di_r, _, rows_r, _, _, ssem = st
    pltpu.make_async_copy(rows_r, acc_sh.at[di_r], ssem).wait()

  def scale(st):
    _, _, wv_r, rows_r, _, _, _ = st
    for g in range(CH // 16):
      wg = wv_r[pl.ds(g * 16, 16)]
      for i in range(16):
        ws = wg[i]
        e = g * 16 + i
        for j in range(D // 16):
          sl = pl.ds(j * 16, 16)
          rows_r[e, sl] = rows_r[e, sl] * ws

  issue_meta(0, sets[0])
  wait_meta(sets[0])
  issue_gather(sets[0])
  issue_meta(1, sets[1])

  def pair(p, carry):
    for h in range(2):
      k = 2 * p + h
      sA = sets[h]
      sB = sets[1 - h]
      wait_gather(sA)
      scale(sA)
      issue_scatter(sA)
      @pl.when(k + 2 < NCHUNK)
      def _():
        wait_scatter(sA)
        issue_meta(k + 2, sA)
      @pl.when(k + 1 < NCHUNK)
      def _():
        wait_meta(sB)
        issue_gather(sB)
    return carry
  lax.fori_loop(0, NCHUNK // 2, pair, 0)
  wait_scatter(sets[0])
  wait_scatter(sets[1])

  plsc.subcore_barrier()
  for z in range(RPW // ZR):
    rr = r0 + z * ZR
    pltpu.sync_copy(acc_sh.at[pl.ds(rr, ZR)], out_hbm.at[c, pl.ds(rr, ZR)])


_GB = B * 2 // (NC * NS)


@functools.partial(
    pl.kernel,
    out_type=jax.ShapeDtypeStruct((2 * B, D), jnp.float32),
    mesh=_MESH,
    scratch_types=[
        pltpu.VMEM((128,), jnp.int32),
        pltpu.VMEM((128, D), jnp.float32),
        pltpu.SemaphoreType.DMA,
    ],
)
def _sc_gather_rows(x_hbm, idx_hbm, out_hbm, idx_v, rows_v, sem):
  c = lax.axis_index("c")
  s = lax.axis_index("s")
  base = (s * NC + c) * _GB
  for t in range(_GB // 128):
    off = base + t * 128
    pltpu.sync_copy(idx_hbm.at[pl.ds(off, 128)], idx_v)
    pltpu.async_copy(x_hbm.at[idx_v], rows_v, sem).wait()
    pltpu.sync_copy(rows_v, out_hbm.at[pl.ds(off, 128)])


def _wsel_call(w2d, t2d):
  def body(w_ref, t_ref, o_ref):
    w = w_ref[...]
    t = t_ref[...]
    o_ref[0] = jnp.where(t == 0, w, 0.0)
    o_ref[1] = jnp.where(t == 1, w, 0.0)
  return pl.pallas_call(
      body,
      out_shape=jax.ShapeDtypeStruct((2, E // 128, 128), jnp.float32),
  )(w2d, t2d)


RB = 1000


def _tc_layer_call(A, x, relw, rootw, bias, g, b2, do_ln):
  def body(a_ref, x_ref, rw_ref, rootw_ref, bias_ref, g_ref, b2_ref, o_ref):
    y = jnp.dot(a_ref[0], rw_ref[0], preferred_element_type=jnp.float32)
    y = y + jnp.dot(a_ref[1], rw_ref[1], preferred_element_type=jnp.float32)
    y = y + jnp.dot(x_ref[...], rootw_ref[...],
                    preferred_element_type=jnp.float32)
    y = y + bias_ref[...]
    if do_ln:
      y = jnp.maximum(y, 0.0)
      m = jnp.mean(y, axis=-1, keepdims=True)
      yc = y - m
      v = jnp.mean(yc * yc, axis=-1, keepdims=True)
      y = yc * lax.rsqrt(v + 1e-5) * g_ref[...] + b2_ref[...]
    o_ref[...] = y
  return pl.pallas_call(
      body,
      grid=(N // RB,),
      in_specs=[
          pl.BlockSpec((2, RB, D), lambda i: (0, i, 0)),
          pl.BlockSpec((RB, D), lambda i: (i, 0)),
          pl.BlockSpec((2, D, D), lambda i: (0, 0, 0)),
          pl.BlockSpec((D, D), lambda i: (0, 0)),
          pl.BlockSpec((1, D), lambda i: (0, 0)),
          pl.BlockSpec((1, D), lambda i: (0, 0)),
          pl.BlockSpec((1, D), lambda i: (0, 0)),
      ],
      out_specs=pl.BlockSpec((RB, D), lambda i: (i, 0)),
      out_shape=jax.ShapeDtypeStruct((N, D), jnp.float32),
  )(A, x, relw, rootw, bias, g, b2)


def _tc_head_call(ui, w0, b0, w1, b1, w2, b2, owt, ob):
  def body(ui_ref, w0_ref, b0_ref, w1_ref, b1_ref, w2_ref, b2_ref,
           ow_ref, ob_ref, o_ref):
    u = ui_ref[:B]
    it = ui_ref[B:]
    h = (jnp.dot(u, w0_ref[:D], preferred_element_type=jnp.float32)
         + jnp.dot(it, w0_ref[D:], preferred_element_type=jnp.float32)
         + b0_ref[...])
    h = jnp.maximum(h, 0.0)
    h = jnp.maximum(
        jnp.dot(h, w1_ref[...], preferred_element_type=jnp.float32)
        + b1_ref[...], 0.0)
    h = jnp.maximum(
        jnp.dot(h, w2_ref[...], preferred_element_type=jnp.float32)
        + b2_ref[...], 0.0)
    nu = jnp.maximum(jnp.sqrt(jnp.sum(u * u, axis=-1, keepdims=True)), 1e-12)
    ni = jnp.maximum(jnp.sqrt(jnp.sum(it * it, axis=-1, keepdims=True)),
                     1e-12)
    gmf = (u / nu) * (it / ni)
    logit = (jnp.sum(gmf * ow_ref[:, :D], axis=-1, keepdims=True)
             + jnp.sum(h * ow_ref[:, D:], axis=-1, keepdims=True)
             + ob_ref[...])
    o_ref[...] = jax.nn.sigmoid(logit)
  return pl.pallas_call(
      body,
      out_shape=jax.ShapeDtypeStruct((B, 1), jnp.float32),
  )(ui, w0, b0, w1, b1, w2, b2, owt, ob)


def kernel(user_indices, item_indices, edge_index, edge_type, edge_weight,
           emb_table, rel_w0, rel_w1, rel_w2, root_w0, root_w1, root_w2,
           bias0, bias1, bias2, ln1_g, ln1_b, ln2_g, ln2_b,
           mlp_w0, mlp_b0, mlp_w1, mlp_b1, mlp_w2, mlp_b2, out_w, out_b):
  src1 = edge_index[0].astype(jnp.int32)
  dst1 = edge_index[1].astype(jnp.int32)
  t2d = edge_type.astype(jnp.int32).reshape(E // 128, 128)
  w2d = edge_weight.reshape(E // 128, 128)
  wsel = _wsel_call(w2d, t2d).reshape(NC * E)

  x = emb_table
  layers = [
      (rel_w0, root_w0, bias0, ln1_g, ln1_b, True),
      (rel_w1, root_w1, bias1, ln2_g, ln2_b, True),
      (rel_w2, root_w2, bias2, ln2_g, ln2_b, False),
  ]
  for relw, rootw, bias, g, b2, do_ln in layers:
    A = _sc_aggregate(x, src1, dst1, wsel)
    x = _tc_layer_call(A, x, relw, rootw.reshape(D, D),
                       bias.reshape(1, D), g.reshape(1, D),
                       b2.reshape(1, D), do_ln)

  idx = jnp.concatenate([user_indices, item_indices]).astype(jnp.int32)
  ui = _sc_gather_rows(x, idx)
  out = _tc_head_call(
      ui, mlp_w0, mlp_b0.reshape(1, -1), mlp_w1, mlp_b1.reshape(1, -1),
      mlp_w2, mlp_b2.reshape(1, -1), out_w.reshape(1, -1),
      out_b.reshape(1, 1))
  return out.reshape(B)
